# baseline (speedup 1.0000x reference)
import jax
import jax.numpy as jnp
from jax.experimental import pallas as pl
from jax.experimental.pallas import tpu as pltpu

_B, _T, _D = 4, 8192, 1024
_BT = 512


def _body(p_ref, o_ref):
  o_ref[0] = p_ref[...]


@jax.jit
def kernel(x, params):
  del x
  return pl.pallas_call(
      _body,
      grid=(_T // _BT, _B),
      in_specs=[pl.BlockSpec((_BT, _D), lambda t, n: (t, 0))],
      out_specs=pl.BlockSpec((1, _BT, _D), lambda t, n: (n, t, 0)),
      out_shape=jax.ShapeDtypeStruct((_B, _T, _D), jnp.float32),
  )(params)


# speedup vs baseline: 1.0258x; 1.0258x over previous
import functools

import jax
import jax.numpy as jnp
from jax import lax
from jax.experimental import pallas as pl
from jax.experimental.pallas import tpu as pltpu
from jax.experimental.pallas import tpu_sc as plsc

_B, _T, _D = 4, 8192, 1024
_NC, _NS = 2, 16
_NW = _NC * _NS
_RPW = _T // _NW
_CH = 32
_NCHUNK = _RPW // _CH


def _make_sc_broadcast():
  mesh = plsc.VectorSubcoreMesh(core_axis_name="c", subcore_axis_name="s")

  @functools.partial(
      pl.kernel,
      out_type=jax.ShapeDtypeStruct((_B, _T, _D), jnp.float32),
      mesh=mesh,
      scratch_types=[
          pltpu.VMEM((_CH, _D), jnp.float32),
          pltpu.VMEM((_CH, _D), jnp.float32),
          pltpu.SemaphoreType.DMA,
          pltpu.SemaphoreType.DMA,
      ],
  )
  def body(params_hbm, out_hbm, buf0, buf1, gsem, ssem):
    bufs = (buf0, buf1)
    wid = lax.axis_index("s") * _NC + lax.axis_index("c")
    base0 = wid * _RPW
    gathers = [None, None]
    scatters = [None, None]
    gathers[0] = pltpu.async_copy(
        params_hbm.at[pl.ds(base0, _CH)], bufs[0], gsem)
    for k in range(_NCHUNK):
      b = k % 2
      gathers[b].wait()
      if k + 1 < _NCHUNK:
        nb = (k + 1) % 2
        if scatters[nb] is not None:
          for cp in scatters[nb]:
            cp.wait()
          scatters[nb] = None
        gathers[nb] = pltpu.async_copy(
            params_hbm.at[pl.ds(base0 + (k + 1) * _CH, _CH)], bufs[nb], gsem)
      scatters[b] = [
          pltpu.async_copy(
              bufs[b], out_hbm.at[n, pl.ds(base0 + k * _CH, _CH)], ssem)
          for n in range(_B)
      ]
    for b in range(2):
      if scatters[b] is not None:
        for cp in scatters[b]:
          cp.wait()

  return body


_sc_broadcast = _make_sc_broadcast()


@jax.jit
def kernel(x, params):
  del x
  return _sc_broadcast(params)
